# baseline (speedup 1.0000x reference)
import jax
import jax.numpy as jnp
from jax.experimental import pallas as pl

K = 20
OUT_CH = 64


def _copy_body(h_ref, o_ref):
    o_ref[...] = h_ref[...]


def kernel(x, W, gamma, beta):
    B, C, N = x.shape
    inner = -2.0 * jnp.matmul(jnp.transpose(x, (0, 2, 1)), x)
    xx = jnp.sum(x ** 2, axis=1, keepdims=True)
    pd = -xx - inner - jnp.transpose(xx, (0, 2, 1))
    idx = jax.lax.top_k(pd, K)[1]
    xt = jnp.transpose(x, (0, 2, 1))
    feature = jax.vmap(lambda pts, ind: pts[ind])(xt, idx)
    xc = jnp.broadcast_to(xt[:, :, None, :], (B, N, K, C))
    feature = jnp.concatenate([feature - xc, xc], axis=3)
    feature = jnp.transpose(feature, (0, 3, 1, 2))
    h = jnp.einsum('oc,bcnk->bonk', W, feature)
    mean = jnp.mean(h, axis=(0, 2, 3), keepdims=True)
    var = jnp.var(h, axis=(0, 2, 3), keepdims=True)
    h = (h - mean) / jnp.sqrt(var + 1e-5)
    h = h * gamma[None, :, None, None] + beta[None, :, None, None]
    h = jnp.where(h >= 0, h, 0.2 * h)
    out = jnp.max(h, axis=3)
    return pl.pallas_call(
        _copy_body,
        out_shape=jax.ShapeDtypeStruct((B, OUT_CH, N), jnp.float32),
        grid=(B,),
        in_specs=[pl.BlockSpec((1, OUT_CH, N), lambda b: (b, 0, 0))],
        out_specs=pl.BlockSpec((1, OUT_CH, N), lambda b: (b, 0, 0)),
    )(out)


# speedup vs baseline: 11.4880x; 11.4880x over previous
import functools

import numpy as np

import jax
import jax.numpy as jnp
from jax import lax
from jax.experimental import pallas as pl
from jax.experimental.pallas import tpu as pltpu
from jax.experimental.pallas import tpu_sc as plsc

K = 20
OUT_CH = 64
B = 16
N = 2048
NEGBIG = -3.0e38


def _const16(vals, dtype=jnp.int32):
    return jnp.asarray(np.asarray(vals), dtype)


def _shuf(v, idx_arr):
    return lax.gather(
        v, idx_arr.reshape(16, 1),
        lax.GatherDimensionNumbers(offset_dims=(), collapsed_slice_dims=(0,),
                                   start_index_map=(0,)),
        slice_sizes=(1,), mode=lax.GatherScatterMode.PROMISE_IN_BOUNDS)


def _pc(mask):
    return plsc.all_reduce_population_count(mask)


def _sorta(v):
    return lax.sort(v)


def _rev(v):
    return lax.rev(v, (0,))


def _merge_split(a_sorted, b_sorted):
    rb = _rev(b_sorted)
    return jnp.maximum(a_sorted, rb), jnp.minimum(a_sorted, rb)


def _sc_body(x_hbm, wc_hbm, mm_hbm, stats_hbm,
             x0_v, x1_v, x2_v, y0_v, y1_v, y2_v, nxx_v, drow_v,
             candv_v, candi_v, selidx_v, outbuf_v, statsbuf_v, wc_v):
    wid = lax.axis_index("s") * 2 + lax.axis_index("c")
    b = wid // 2
    half = wid % 2
    base = half * 1024

    izero16 = jnp.full((16,), wid * 0)
    zero16 = izero16.astype(jnp.float32)
    lanes = plsc.cumsum(izero16 + 1) - 1
    lane_lt4 = lanes < 4
    negbig16 = zero16 + NEGBIG

    def bcast(v, j):
        return _shuf(v, izero16 + j)

    pltpu.sync_copy(x_hbm.at[pl.ds((b * 3 + 0) * N, N)], x0_v)
    pltpu.sync_copy(x_hbm.at[pl.ds((b * 3 + 1) * N, N)], x1_v)
    pltpu.sync_copy(x_hbm.at[pl.ds((b * 3 + 2) * N, N)], x2_v)
    pltpu.sync_copy(wc_hbm, wc_v)

    def prep(c, _):
        s = pl.ds(c * 16, 16)
        a0, a1, a2 = x0_v[s], x1_v[s], x2_v[s]
        def trunc_bf16(a):
            u = plsc.bitcast(a, jnp.uint32)
            u = u + jnp.uint32(0x7FFF) + ((u >> 16) & jnp.uint32(1))
            return plsc.bitcast(u & jnp.uint32(0xFFFF0000), jnp.float32)

        b0 = trunc_bf16(a0)
        b1 = trunc_bf16(a1)
        b2 = trunc_bf16(a2)
        y0_v[s] = b0 + b0
        y1_v[s] = b1 + b1
        y2_v[s] = b2 + b2
        sq = a0 * a0 + a1 * a1 + a2 * a2
        nxx_v[s] = -sq
        return 0

    lax.fori_loop(0, 128, prep, 0)

    def xstat(c, acc):
        s = pl.ds(base + c * 16, 16)
        a0, a1, a2 = x0_v[s], x1_v[s], x2_v[s]
        return (acc[0] + a0, acc[1] + a1, acc[2] + a2,
                acc[3] + a0 * a0, acc[4] + a1 * a1, acc[5] + a2 * a2,
                acc[6] + a0 * a1, acc[7] + a0 * a2, acc[8] + a1 * a2)

    xacc = lax.fori_loop(0, 64, xstat, (zero16,) * 9)

    wcg = [[wc_v[pl.ds(c * 64 + g * 16, 16)] for g in range(4)]
           for c in range(3)]

    def row_pipeline(n, carry):
        nsp = jnp.full((16,), n, jnp.int32)
        x0n = plsc.load_gather(x0_v, [nsp])
        x1n = plsc.load_gather(x1_v, [nsp])
        x2n = plsc.load_gather(x2_v, [nsp])
        nxxn = plsc.load_gather(nxx_v, [nsp])
        x0nb = plsc.load_gather(y0_v, [nsp]) * 0.5
        x1nb = plsc.load_gather(y1_v, [nsp]) * 0.5
        x2nb = plsc.load_gather(y2_v, [nsp]) * 0.5

        def bchunk(c, car):
            m1, m2 = car
            s = pl.ds(c * 16, 16)
            acc = y0_v[s] * x0nb
            acc = y1_v[s] * x1nb + acc
            acc = y2_v[s] * x2nb + acc
            d = (acc + nxx_v[s]) + nxxn
            drow_v[s] = d
            m2 = jnp.maximum(m2, jnp.minimum(m1, d))
            m1 = jnp.maximum(m1, d)
            return m1, m2

        m1, m2 = lax.fori_loop(0, 128, bchunk, (negbig16, negbig16))

        s1 = _sorta(m1)
        s2 = _sorta(m2)
        _, lo = _merge_split(s1, s2)
        thr = _sorta(lo)[12]

        candv_v[pl.ds(0, 16)] = negbig16
        candv_v[pl.ds(16, 16)] = negbig16
        candv_v[pl.ds(32, 16)] = negbig16
        candv_v[pl.ds(48, 16)] = negbig16

        def cchunk(c, off):
            s = pl.ds(c * 16, 16)
            d = drow_v[s]
            msk = d >= thr
            plsc.store_compressed(candv_v.at[pl.ds(off, 16)], d, mask=msk)
            idxs = lanes + c * 16
            plsc.store_compressed(candi_v.at[pl.ds(off, 16)], idxs, mask=msk)
            return jnp.minimum(off + _pc(msk)[0], 48)

        lax.fori_loop(0, 128, cchunk, 0)

        c0 = candv_v[pl.ds(0, 16)]
        c1 = candv_v[pl.ds(16, 16)]
        c2 = candv_v[pl.ds(32, 16)]
        c3 = candv_v[pl.ds(48, 16)]
        s0_, s1_, s2_, s3_ = _sorta(c0), _sorta(c1), _sorta(c2), _sorta(c3)
        hi1, lo1 = _merge_split(s0_, s1_)
        hi2, lo2 = _merge_split(s2_, s3_)
        h1s, h2s, l1s, l2s = _sorta(hi1), _sorta(hi2), _sorta(lo1), _sorta(lo2)
        _, alo = _merge_split(h1s, h2s)
        bhi, _ = _merge_split(l1s, l2s)
        chi, _ = _merge_split(_sorta(alo), _sorta(bhi))
        t20 = _sorta(chi)[12]

        gt = [c0 > t20, c1 > t20, c2 > t20, c3 > t20]
        eq = [c0 == t20, c1 == t20, c2 == t20, c3 == t20]
        cnt_gt = _pc(gt[0]) + _pc(gt[1]) + _pc(gt[2]) + _pc(gt[3])
        r = 20 - cnt_gt
        sel = []
        base_eq = izero16
        for i in range(4):
            cum = plsc.cumsum(eq[i].astype(jnp.int32)) + base_eq
            sel.append(gt[i] | (eq[i] & (cum <= r)))
            base_eq = base_eq + _pc(eq[i])

        selidx_v[pl.ds(16, 16)] = izero16
        off = 0
        for i in range(4):
            ci = candi_v[pl.ds(i * 16, 16)]
            plsc.store_compressed(selidx_v.at[pl.ds(off, 16)], ci, mask=sel[i])
            off = off + _pc(sel[i])[0]

        iv0 = selidx_v[pl.ds(0, 16)]
        iv1 = selidx_v[pl.ds(16, 16)]
        g0a = plsc.load_gather(x0_v, [iv0])
        g1a = plsc.load_gather(x1_v, [iv0])
        g2a = plsc.load_gather(x2_v, [iv0])
        g0b = plsc.load_gather(x0_v, [iv1])
        g1b = plsc.load_gather(x1_v, [iv1])
        g2b = plsc.load_gather(x2_v, [iv1])

        mx = [None] * 4
        mn = [None] * 4
        for j in range(20):
            if j < 16:
                xj0, xj1, xj2 = bcast(g0a, j), bcast(g1a, j), bcast(g2a, j)
            else:
                xj0, xj1, xj2 = (bcast(g0b, j - 16), bcast(g1b, j - 16),
                                 bcast(g2b, j - 16))
            for g in range(4):
                e = wcg[0][g] * xj0 + wcg[1][g] * xj1 + wcg[2][g] * xj2
                if j == 0:
                    mx[g] = e
                    mn[g] = e
                else:
                    mx[g] = jnp.maximum(mx[g], e)
                    mn[g] = jnp.minimum(mn[g], e)

        obase = ((n - base) % 256) * 128
        for g in range(4):
            outbuf_v[pl.ds(obase + g * 16, 16)] = mx[g]
            outbuf_v[pl.ds(obase + 64 + g * 16, 16)] = mn[g]

        m0a = jnp.where(lane_lt4, g0b, 0.0)
        m1a = jnp.where(lane_lt4, g1b, 0.0)
        m2a = jnp.where(lane_lt4, g2b, 0.0)
        t0 = g0a + m0a
        t1 = g1a + m1a
        t2 = g2a + m2a
        (sx0, sx1, sx2, q00, q11, q22, q01, q02, q12,
         p00, p01, p02, p10, p11, p12, p20, p21, p22) = carry
        sx0 = sx0 + t0
        sx1 = sx1 + t1
        sx2 = sx2 + t2
        q00 = q00 + g0a * g0a + m0a * m0a
        q11 = q11 + g1a * g1a + m1a * m1a
        q22 = q22 + g2a * g2a + m2a * m2a
        q01 = q01 + g0a * g1a + m0a * m1a
        q02 = q02 + g0a * g2a + m0a * m2a
        q12 = q12 + g1a * g2a + m1a * m2a
        p00 = p00 + x0n * t0
        p01 = p01 + x0n * t1
        p02 = p02 + x0n * t2
        p10 = p10 + x1n * t0
        p11 = p11 + x1n * t1
        p12 = p12 + x1n * t2
        p20 = p20 + x2n * t0
        p21 = p21 + x2n * t1
        p22 = p22 + x2n * t2
        return (sx0, sx1, sx2, q00, q11, q22, q01, q02, q12,
                p00, p01, p02, p10, p11, p12, p20, p21, p22)

    def seg_body(seg, carry):
        def row_body(r, car):
            return row_pipeline(base + seg * 256 + r, car)

        carry = lax.fori_loop(0, 256, row_body, carry)
        off = (b * N + base + seg * 256) * 128
        pltpu.sync_copy(outbuf_v, mm_hbm.at[pl.ds(off, 256 * 128)])
        return carry

    stats0 = (zero16,) * 18
    stats = lax.fori_loop(0, 4, seg_body, stats0)

    allstats = list(stats) + list(xacc)
    for i, vec in enumerate(allstats):
        statsbuf_v[pl.ds(i * 16, 16)] = vec
    for i in range(27, 32):
        statsbuf_v[pl.ds(i * 16, 16)] = zero16
    pltpu.sync_copy(statsbuf_v, stats_hbm.at[pl.ds(wid * 512, 512)])


def _sc_knn(xflat, wcflat):
    mesh = plsc.VectorSubcoreMesh(core_axis_name="c", subcore_axis_name="s")
    kern = functools.partial(
        pl.kernel,
        mesh=mesh,
        compiler_params=pltpu.CompilerParams(needs_layout_passes=False),
        out_type=[
            jax.ShapeDtypeStruct((B * N * 128,), jnp.float32),
            jax.ShapeDtypeStruct((32 * 512,), jnp.float32),
        ],
        scratch_types=[
            pltpu.VMEM((N,), jnp.float32),
            pltpu.VMEM((N,), jnp.float32),
            pltpu.VMEM((N,), jnp.float32),
            pltpu.VMEM((N,), jnp.float32),
            pltpu.VMEM((N,), jnp.float32),
            pltpu.VMEM((N,), jnp.float32),
            pltpu.VMEM((N,), jnp.float32),
            pltpu.VMEM((N,), jnp.float32),
            pltpu.VMEM((64,), jnp.float32),
            pltpu.VMEM((64,), jnp.int32),
            pltpu.VMEM((32,), jnp.int32),
            pltpu.VMEM((256 * 128,), jnp.float32),
            pltpu.VMEM((512,), jnp.float32),
            pltpu.VMEM((192,), jnp.float32),
        ],
    )(_sc_body)
    return kern(xflat, wcflat)


def _tc_stats_body(stats_ref, wp_ref, gamma_ref, beta_ref, out_ref):
    s = [jnp.sum(stats_ref[:, pl.ds(i * 16, 16)]) for i in range(27)]
    (sx0, sx1, sx2, q00, q11, q22, q01, q02, q12,
     p00, p01, p02, p10, p11, p12, p20, p21, p22,
     xs0, xs1, xs2, e00, e11, e22, e01, e02, e12) = s
    w1 = [wp_ref[pl.ds(c, 1), :] for c in range(3)]
    wd = [wp_ref[pl.ds(3 + c, 1), :] for c in range(3)]
    kf = jnp.float32(K)
    sum_h = (w1[0] * sx0 + w1[1] * sx1 + w1[2] * sx2
             + kf * (wd[0] * xs0 + wd[1] * xs1 + wd[2] * xs2))
    q_term = (w1[0] * w1[0] * q00 + w1[1] * w1[1] * q11 + w1[2] * w1[2] * q22
              + 2.0 * (w1[0] * w1[1] * q01 + w1[0] * w1[2] * q02
                       + w1[1] * w1[2] * q12))
    cross = (wd[0] * w1[0] * p00 + wd[0] * w1[1] * p01 + wd[0] * w1[2] * p02
             + wd[1] * w1[0] * p10 + wd[1] * w1[1] * p11 + wd[1] * w1[2] * p12
             + wd[2] * w1[0] * p20 + wd[2] * w1[1] * p21 + wd[2] * w1[2] * p22)
    v2 = (wd[0] * wd[0] * e00 + wd[1] * wd[1] * e11 + wd[2] * wd[2] * e22
          + 2.0 * (wd[0] * wd[1] * e01 + wd[0] * wd[2] * e02
                   + wd[1] * wd[2] * e12))
    sum_h2 = q_term + 2.0 * cross + kf * v2
    cnt = jnp.float32(B * N * K)
    mean = sum_h / cnt
    var = sum_h2 / cnt - mean * mean
    rstd = 1.0 / jnp.sqrt(var + 1e-5)
    scale = gamma_ref[...] * rstd
    shift = beta_ref[...] - mean * scale
    pad = jnp.zeros((6, 64), jnp.float32)
    out_ref[...] = jnp.concatenate([scale, shift, pad], axis=0)


def _tc_apply_body(mm_ref, xt_ref, wdp_ref, ss_ref, out_ref):
    xb = xt_ref[0]
    vt = jnp.dot(xb, wdp_ref[...], preferred_element_type=jnp.float32)
    mm = mm_ref[0]
    mx = lax.slice(mm, (0, 0), (512, 64))
    mn = lax.slice(mm, (0, 64), (512, 128))
    scale = ss_ref[pl.ds(0, 1), :]
    shift = ss_ref[pl.ds(1, 1), :]
    selv = jnp.where(scale >= 0.0, mx, mn)
    h = (selv + vt) * scale + shift
    out_ref[0] = jnp.where(h >= 0.0, h, 0.2 * h)


def kernel(x, W, gamma, beta):
    W1 = W[:, :3]
    Wd = W[:, 3:] - W1
    wcflat = jnp.reshape(jnp.transpose(W1), (-1,))
    xflat = jnp.reshape(x, (-1,))
    mm_flat, stats_flat = _sc_knn(xflat, wcflat)

    wp = jnp.zeros((8, 64), jnp.float32)
    wp = wp.at[0:3].set(jnp.transpose(W1)).at[3:6].set(jnp.transpose(Wd))
    ss = pl.pallas_call(
        _tc_stats_body,
        out_shape=jax.ShapeDtypeStruct((8, 64), jnp.float32),
        in_specs=[
            pl.BlockSpec((32, 512), lambda: (0, 0)),
            pl.BlockSpec((8, 64), lambda: (0, 0)),
            pl.BlockSpec((1, 64), lambda: (0, 0)),
            pl.BlockSpec((1, 64), lambda: (0, 0)),
        ],
        out_specs=pl.BlockSpec((8, 64), lambda: (0, 0)),
    )(stats_flat.reshape(32, 512), wp, gamma.reshape(1, 64),
      beta.reshape(1, 64))

    xt_pad = jnp.zeros((B, N, 8), jnp.float32)
    xt_pad = xt_pad.at[:, :, :3].set(jnp.transpose(x, (0, 2, 1)))
    wdp = jnp.zeros((8, 64), jnp.float32).at[0:3].set(jnp.transpose(Wd))

    out_nt = pl.pallas_call(
        _tc_apply_body,
        out_shape=jax.ShapeDtypeStruct((B, N, OUT_CH), jnp.float32),
        grid=(B, N // 512),
        in_specs=[
            pl.BlockSpec((1, 512, 128), lambda b, j: (b, j, 0)),
            pl.BlockSpec((1, 512, 8), lambda b, j: (b, j, 0)),
            pl.BlockSpec((8, 64), lambda b, j: (0, 0)),
            pl.BlockSpec((8, 64), lambda b, j: (0, 0)),
        ],
        out_specs=pl.BlockSpec((1, 512, OUT_CH), lambda b, j: (b, j, 0)),
    )(mm_flat.reshape(B, N, 128), xt_pad, wdp, ss)

    return jnp.transpose(out_nt, (0, 2, 1))


# speedup vs baseline: 12.1327x; 1.0561x over previous
import functools

import numpy as np

import jax
import jax.numpy as jnp
from jax import lax
from jax.experimental import pallas as pl
from jax.experimental.pallas import tpu as pltpu
from jax.experimental.pallas import tpu_sc as plsc

K = 20
OUT_CH = 64
B = 16
N = 2048
NEGBIG = -3.0e38


def _const16(vals, dtype=jnp.int32):
    return jnp.asarray(np.asarray(vals), dtype)


def _shuf(v, idx_arr):
    return lax.gather(
        v, idx_arr.reshape(16, 1),
        lax.GatherDimensionNumbers(offset_dims=(), collapsed_slice_dims=(0,),
                                   start_index_map=(0,)),
        slice_sizes=(1,), mode=lax.GatherScatterMode.PROMISE_IN_BOUNDS)


def _pc(mask):
    return plsc.all_reduce_population_count(mask)


def _sorta(v):
    return lax.sort(v)


def _rev(v):
    return lax.rev(v, (0,))


def _merge_split(a_sorted, b_sorted):
    rb = _rev(b_sorted)
    return jnp.maximum(a_sorted, rb), jnp.minimum(a_sorted, rb)


def _sc_body(x_hbm, wc_hbm, mm_hbm, stats_hbm,
             x0_v, x1_v, x2_v, y0_v, y1_v, y2_v, nxx_v, drow_v,
             candv_v, candi_v, selidx_v, outbuf_v, statsbuf_v, wc_v):
    wid = lax.axis_index("s") * 2 + lax.axis_index("c")
    b = wid // 2
    half = wid % 2
    base = half * 1024

    izero16 = jnp.full((16,), wid * 0)
    zero16 = izero16.astype(jnp.float32)
    lanes = plsc.cumsum(izero16 + 1) - 1
    lane_lt4 = lanes < 4
    negbig16 = zero16 + NEGBIG

    def bcast(v, j):
        return _shuf(v, izero16 + j)

    pltpu.sync_copy(x_hbm.at[pl.ds((b * 3 + 0) * N, N)], x0_v)
    pltpu.sync_copy(x_hbm.at[pl.ds((b * 3 + 1) * N, N)], x1_v)
    pltpu.sync_copy(x_hbm.at[pl.ds((b * 3 + 2) * N, N)], x2_v)
    pltpu.sync_copy(wc_hbm, wc_v)

    def prep(c, _):
        s = pl.ds(c * 16, 16)
        a0, a1, a2 = x0_v[s], x1_v[s], x2_v[s]
        def trunc_bf16(a):
            u = plsc.bitcast(a, jnp.uint32)
            u = u + jnp.uint32(0x7FFF) + ((u >> 16) & jnp.uint32(1))
            return plsc.bitcast(u & jnp.uint32(0xFFFF0000), jnp.float32)

        b0 = trunc_bf16(a0)
        b1 = trunc_bf16(a1)
        b2 = trunc_bf16(a2)
        y0_v[s] = b0 + b0
        y1_v[s] = b1 + b1
        y2_v[s] = b2 + b2
        sq = a0 * a0 + a1 * a1 + a2 * a2
        nxx_v[s] = -sq
        return 0

    lax.fori_loop(0, 128, prep, 0)

    def xstat(c, acc):
        s = pl.ds(base + c * 16, 16)
        a0, a1, a2 = x0_v[s], x1_v[s], x2_v[s]
        return (acc[0] + a0, acc[1] + a1, acc[2] + a2,
                acc[3] + a0 * a0, acc[4] + a1 * a1, acc[5] + a2 * a2,
                acc[6] + a0 * a1, acc[7] + a0 * a2, acc[8] + a1 * a2)

    xacc = lax.fori_loop(0, 64, xstat, (zero16,) * 9)

    wcg = [[wc_v[pl.ds(c * 64 + g * 16, 16)] for g in range(4)]
           for c in range(3)]

    def row_pipeline(n, carry):
        nsp = jnp.full((16,), n, jnp.int32)
        x0n = plsc.load_gather(x0_v, [nsp])
        x1n = plsc.load_gather(x1_v, [nsp])
        x2n = plsc.load_gather(x2_v, [nsp])
        nxxn = plsc.load_gather(nxx_v, [nsp])
        x0nb = plsc.load_gather(y0_v, [nsp]) * 0.5
        x1nb = plsc.load_gather(y1_v, [nsp]) * 0.5
        x2nb = plsc.load_gather(y2_v, [nsp]) * 0.5

        def bchunk(c4, car):
            m1, m2 = car
            for u in range(4):
                s = pl.ds(c4 * 64 + u * 16, 16)
                acc = y0_v[s] * x0nb
                acc = y1_v[s] * x1nb + acc
                acc = y2_v[s] * x2nb + acc
                d = (acc + nxx_v[s]) + nxxn
                drow_v[s] = d
                m2 = jnp.maximum(m2, jnp.minimum(m1, d))
                m1 = jnp.maximum(m1, d)
            return m1, m2

        m1, m2 = lax.fori_loop(0, 32, bchunk, (negbig16, negbig16))

        s1 = _sorta(m1)
        s2 = _sorta(m2)
        _, lo = _merge_split(s1, s2)
        thr = _sorta(lo)[12]

        candv_v[pl.ds(0, 16)] = negbig16
        candv_v[pl.ds(16, 16)] = negbig16
        candv_v[pl.ds(32, 16)] = negbig16
        candv_v[pl.ds(48, 16)] = negbig16

        def cchunk(c2, off):
            for u in range(2):
                c = c2 * 2 + u
                s = pl.ds(c * 16, 16)
                d = drow_v[s]
                msk = d >= thr
                plsc.store_compressed(candv_v.at[pl.ds(off, 16)], d, mask=msk)
                idxs = lanes + c * 16
                plsc.store_compressed(candi_v.at[pl.ds(off, 16)], idxs,
                                      mask=msk)
                off = jnp.minimum(off + _pc(msk)[0], 48)
            return off

        lax.fori_loop(0, 64, cchunk, 0)

        c0 = candv_v[pl.ds(0, 16)]
        c1 = candv_v[pl.ds(16, 16)]
        c2 = candv_v[pl.ds(32, 16)]
        c3 = candv_v[pl.ds(48, 16)]
        s0_, s1_, s2_, s3_ = _sorta(c0), _sorta(c1), _sorta(c2), _sorta(c3)
        hi1, lo1 = _merge_split(s0_, s1_)
        hi2, lo2 = _merge_split(s2_, s3_)
        h1s, h2s, l1s, l2s = _sorta(hi1), _sorta(hi2), _sorta(lo1), _sorta(lo2)
        _, alo = _merge_split(h1s, h2s)
        bhi, _ = _merge_split(l1s, l2s)
        chi, _ = _merge_split(_sorta(alo), _sorta(bhi))
        t20 = _sorta(chi)[12]

        gt = [c0 > t20, c1 > t20, c2 > t20, c3 > t20]
        eq = [c0 == t20, c1 == t20, c2 == t20, c3 == t20]
        cnt_gt = _pc(gt[0]) + _pc(gt[1]) + _pc(gt[2]) + _pc(gt[3])
        r = 20 - cnt_gt
        sel = []
        base_eq = izero16
        for i in range(4):
            cum = plsc.cumsum(eq[i].astype(jnp.int32)) + base_eq
            sel.append(gt[i] | (eq[i] & (cum <= r)))
            base_eq = base_eq + _pc(eq[i])

        selidx_v[pl.ds(16, 16)] = izero16
        off = 0
        for i in range(4):
            ci = candi_v[pl.ds(i * 16, 16)]
            plsc.store_compressed(selidx_v.at[pl.ds(off, 16)], ci, mask=sel[i])
            off = off + _pc(sel[i])[0]

        iv0 = selidx_v[pl.ds(0, 16)]
        iv1 = selidx_v[pl.ds(16, 16)]
        g0a = plsc.load_gather(x0_v, [iv0])
        g1a = plsc.load_gather(x1_v, [iv0])
        g2a = plsc.load_gather(x2_v, [iv0])
        g0b = plsc.load_gather(x0_v, [iv1])
        g1b = plsc.load_gather(x1_v, [iv1])
        g2b = plsc.load_gather(x2_v, [iv1])

        mx = [None] * 4
        mn = [None] * 4
        for j in range(20):
            if j < 16:
                xj0, xj1, xj2 = bcast(g0a, j), bcast(g1a, j), bcast(g2a, j)
            else:
                xj0, xj1, xj2 = (bcast(g0b, j - 16), bcast(g1b, j - 16),
                                 bcast(g2b, j - 16))
            for g in range(4):
                e = wcg[0][g] * xj0 + wcg[1][g] * xj1 + wcg[2][g] * xj2
                if j == 0:
                    mx[g] = e
                    mn[g] = e
                else:
                    mx[g] = jnp.maximum(mx[g], e)
                    mn[g] = jnp.minimum(mn[g], e)

        obase = ((n - base) % 256) * 128
        for g in range(4):
            outbuf_v[pl.ds(obase + g * 16, 16)] = mx[g]
            outbuf_v[pl.ds(obase + 64 + g * 16, 16)] = mn[g]

        m0a = jnp.where(lane_lt4, g0b, 0.0)
        m1a = jnp.where(lane_lt4, g1b, 0.0)
        m2a = jnp.where(lane_lt4, g2b, 0.0)
        t0 = g0a + m0a
        t1 = g1a + m1a
        t2 = g2a + m2a
        (sx0, sx1, sx2, q00, q11, q22, q01, q02, q12,
         p00, p01, p02, p10, p11, p12, p20, p21, p22) = carry
        sx0 = sx0 + t0
        sx1 = sx1 + t1
        sx2 = sx2 + t2
        q00 = q00 + g0a * g0a + m0a * m0a
        q11 = q11 + g1a * g1a + m1a * m1a
        q22 = q22 + g2a * g2a + m2a * m2a
        q01 = q01 + g0a * g1a + m0a * m1a
        q02 = q02 + g0a * g2a + m0a * m2a
        q12 = q12 + g1a * g2a + m1a * m2a
        p00 = p00 + x0n * t0
        p01 = p01 + x0n * t1
        p02 = p02 + x0n * t2
        p10 = p10 + x1n * t0
        p11 = p11 + x1n * t1
        p12 = p12 + x1n * t2
        p20 = p20 + x2n * t0
        p21 = p21 + x2n * t1
        p22 = p22 + x2n * t2
        return (sx0, sx1, sx2, q00, q11, q22, q01, q02, q12,
                p00, p01, p02, p10, p11, p12, p20, p21, p22)

    def seg_body(seg, carry):
        def row_body(r, car):
            return row_pipeline(base + seg * 256 + r, car)

        carry = lax.fori_loop(0, 256, row_body, carry)
        off = (b * N + base + seg * 256) * 128
        pltpu.sync_copy(outbuf_v, mm_hbm.at[pl.ds(off, 256 * 128)])
        return carry

    stats0 = (zero16,) * 18
    stats = lax.fori_loop(0, 4, seg_body, stats0)

    allstats = list(stats) + list(xacc)
    for i, vec in enumerate(allstats):
        statsbuf_v[pl.ds(i * 16, 16)] = vec
    for i in range(27, 32):
        statsbuf_v[pl.ds(i * 16, 16)] = zero16
    pltpu.sync_copy(statsbuf_v, stats_hbm.at[pl.ds(wid * 512, 512)])


def _sc_knn(xflat, wcflat):
    mesh = plsc.VectorSubcoreMesh(core_axis_name="c", subcore_axis_name="s")
    kern = functools.partial(
        pl.kernel,
        mesh=mesh,
        compiler_params=pltpu.CompilerParams(needs_layout_passes=False),
        out_type=[
            jax.ShapeDtypeStruct((B * N * 128,), jnp.float32),
            jax.ShapeDtypeStruct((32 * 512,), jnp.float32),
        ],
        scratch_types=[
            pltpu.VMEM((N,), jnp.float32),
            pltpu.VMEM((N,), jnp.float32),
            pltpu.VMEM((N,), jnp.float32),
            pltpu.VMEM((N,), jnp.float32),
            pltpu.VMEM((N,), jnp.float32),
            pltpu.VMEM((N,), jnp.float32),
            pltpu.VMEM((N,), jnp.float32),
            pltpu.VMEM((N,), jnp.float32),
            pltpu.VMEM((64,), jnp.float32),
            pltpu.VMEM((64,), jnp.int32),
            pltpu.VMEM((32,), jnp.int32),
            pltpu.VMEM((256 * 128,), jnp.float32),
            pltpu.VMEM((512,), jnp.float32),
            pltpu.VMEM((192,), jnp.float32),
        ],
    )(_sc_body)
    return kern(xflat, wcflat)


def _tc_stats_body(stats_ref, wp_ref, gamma_ref, beta_ref, out_ref):
    s = [jnp.sum(stats_ref[:, pl.ds(i * 16, 16)]) for i in range(27)]
    (sx0, sx1, sx2, q00, q11, q22, q01, q02, q12,
     p00, p01, p02, p10, p11, p12, p20, p21, p22,
     xs0, xs1, xs2, e00, e11, e22, e01, e02, e12) = s
    w1 = [wp_ref[pl.ds(c, 1), :] for c in range(3)]
    wd = [wp_ref[pl.ds(3 + c, 1), :] for c in range(3)]
    kf = jnp.float32(K)
    sum_h = (w1[0] * sx0 + w1[1] * sx1 + w1[2] * sx2
             + kf * (wd[0] * xs0 + wd[1] * xs1 + wd[2] * xs2))
    q_term = (w1[0] * w1[0] * q00 + w1[1] * w1[1] * q11 + w1[2] * w1[2] * q22
              + 2.0 * (w1[0] * w1[1] * q01 + w1[0] * w1[2] * q02
                       + w1[1] * w1[2] * q12))
    cross = (wd[0] * w1[0] * p00 + wd[0] * w1[1] * p01 + wd[0] * w1[2] * p02
             + wd[1] * w1[0] * p10 + wd[1] * w1[1] * p11 + wd[1] * w1[2] * p12
             + wd[2] * w1[0] * p20 + wd[2] * w1[1] * p21 + wd[2] * w1[2] * p22)
    v2 = (wd[0] * wd[0] * e00 + wd[1] * wd[1] * e11 + wd[2] * wd[2] * e22
          + 2.0 * (wd[0] * wd[1] * e01 + wd[0] * wd[2] * e02
                   + wd[1] * wd[2] * e12))
    sum_h2 = q_term + 2.0 * cross + kf * v2
    cnt = jnp.float32(B * N * K)
    mean = sum_h / cnt
    var = sum_h2 / cnt - mean * mean
    rstd = 1.0 / jnp.sqrt(var + 1e-5)
    scale = gamma_ref[...] * rstd
    shift = beta_ref[...] - mean * scale
    pad = jnp.zeros((6, 64), jnp.float32)
    out_ref[...] = jnp.concatenate([scale, shift, pad], axis=0)


def _tc_apply_body(mm_ref, xt_ref, wdp_ref, ss_ref, out_ref):
    xb = xt_ref[0]
    vt = jnp.dot(xb, wdp_ref[...], preferred_element_type=jnp.float32)
    mm = mm_ref[0]
    mx = lax.slice(mm, (0, 0), (512, 64))
    mn = lax.slice(mm, (0, 64), (512, 128))
    scale = ss_ref[pl.ds(0, 1), :]
    shift = ss_ref[pl.ds(1, 1), :]
    selv = jnp.where(scale >= 0.0, mx, mn)
    h = (selv + vt) * scale + shift
    out_ref[0] = jnp.where(h >= 0.0, h, 0.2 * h)


def kernel(x, W, gamma, beta):
    W1 = W[:, :3]
    Wd = W[:, 3:] - W1
    wcflat = jnp.reshape(jnp.transpose(W1), (-1,))
    xflat = jnp.reshape(x, (-1,))
    mm_flat, stats_flat = _sc_knn(xflat, wcflat)

    wp = jnp.zeros((8, 64), jnp.float32)
    wp = wp.at[0:3].set(jnp.transpose(W1)).at[3:6].set(jnp.transpose(Wd))
    ss = pl.pallas_call(
        _tc_stats_body,
        out_shape=jax.ShapeDtypeStruct((8, 64), jnp.float32),
        in_specs=[
            pl.BlockSpec((32, 512), lambda: (0, 0)),
            pl.BlockSpec((8, 64), lambda: (0, 0)),
            pl.BlockSpec((1, 64), lambda: (0, 0)),
            pl.BlockSpec((1, 64), lambda: (0, 0)),
        ],
        out_specs=pl.BlockSpec((8, 64), lambda: (0, 0)),
    )(stats_flat.reshape(32, 512), wp, gamma.reshape(1, 64),
      beta.reshape(1, 64))

    xt_pad = jnp.zeros((B, N, 8), jnp.float32)
    xt_pad = xt_pad.at[:, :, :3].set(jnp.transpose(x, (0, 2, 1)))
    wdp = jnp.zeros((8, 64), jnp.float32).at[0:3].set(jnp.transpose(Wd))

    out_nt = pl.pallas_call(
        _tc_apply_body,
        out_shape=jax.ShapeDtypeStruct((B, N, OUT_CH), jnp.float32),
        grid=(B, N // 512),
        in_specs=[
            pl.BlockSpec((1, 512, 128), lambda b, j: (b, j, 0)),
            pl.BlockSpec((1, 512, 8), lambda b, j: (b, j, 0)),
            pl.BlockSpec((8, 64), lambda b, j: (0, 0)),
            pl.BlockSpec((8, 64), lambda b, j: (0, 0)),
        ],
        out_specs=pl.BlockSpec((1, 512, OUT_CH), lambda b, j: (b, j, 0)),
    )(mm_flat.reshape(B, N, 128), xt_pad, wdp, ss)

    return jnp.transpose(out_nt, (0, 2, 1))
